# baseline (speedup 1.0000x reference)
import functools

import jax
import jax.numpy as jnp
from jax import lax
from jax.experimental import pallas as pl
from jax.experimental.pallas import tpu as pltpu
from jax.experimental.pallas import tpu_sc as plsc

VOCAB = 5000
D = 128
H = 256
A = 8
B = 4096
L = 50
L_PAD = 56

_info = plsc.get_sparse_core_info()
NC = _info.num_cores
NS = _info.num_subcores
NW = NC * NS
LANES = _info.num_lanes

B_PER_W = B // NW
CHUNK_ROWS = 2
IDX_PER_CHUNK = CHUNK_ROWS * L_PAD
NCHUNK = B_PER_W // CHUNK_ROWS
DJ = D // LANES

_sc_mesh = plsc.VectorSubcoreMesh(core_axis_name="c", subcore_axis_name="s")


@functools.partial(
    pl.kernel,
    mesh=_sc_mesh,
    out_type=(
        jax.ShapeDtypeStruct((B, D), jnp.float32),
        jax.ShapeDtypeStruct((B, D), jnp.float32),
    ),
    scratch_types=[
        pltpu.VMEM((B_PER_W * L_PAD,), jnp.int32),
        pltpu.VMEM((B_PER_W * L_PAD,), jnp.int32),
        pltpu.VMEM((IDX_PER_CHUNK, D), jnp.float32),
        pltpu.VMEM((IDX_PER_CHUNK, D), jnp.float32),
        pltpu.VMEM((B_PER_W, D), jnp.float32),
        pltpu.SemaphoreType.DMA,
        pltpu.SemaphoreType.DMA,
    ],
)
def _pool(itok_hbm, stok_hbm, temb_hbm, semb_hbm, iout_hbm, sout_hbm,
          idx_i, idx_s, buf0, buf1, pooled, sem0, sem1):
    wid = lax.axis_index("s") * NC + lax.axis_index("c")
    base = wid * B_PER_W
    pltpu.sync_copy(itok_hbm.at[pl.ds(base * L_PAD, B_PER_W * L_PAD)], idx_i)
    pltpu.sync_copy(stok_hbm.at[pl.ds(base * L_PAD, B_PER_W * L_PAD)], idx_s)
    bufs = (buf0, buf1)
    sems = (sem0, sem1)
    inv = jnp.float32(1.0 / L)

    for idx_ref, emb_hbm, out_hbm in ((idx_i, temb_hbm, iout_hbm),
                                      (idx_s, semb_hbm, sout_hbm)):
        pltpu.async_copy(
            emb_hbm.at[idx_ref.at[pl.ds(0, IDX_PER_CHUNK)]], bufs[0], sems[0])

        def outer(kk, carry):
            for bsel in range(2):
                k = kk * 2 + bsel
                buf = bufs[bsel]
                sem = sems[bsel]
                nb = 1 - bsel

                @pl.when(k + 1 < NCHUNK)
                def _fire():
                    off = pl.multiple_of((k + 1) * IDX_PER_CHUNK, 8)
                    pltpu.async_copy(
                        emb_hbm.at[idx_ref.at[pl.ds(off, IDX_PER_CHUNK)]],
                        bufs[nb], sems[nb])

                off_k = pl.multiple_of(k * IDX_PER_CHUNK, 8)
                pltpu.make_async_copy(
                    emb_hbm.at[idx_ref.at[pl.ds(off_k, IDX_PER_CHUNK)]],
                    buf, sem).wait()

                for r in range(CHUNK_ROWS):
                    rb = r * L_PAD

                    def red(l, accs, _rb=rb, _buf=buf):
                        return tuple(
                            accs[j] + _buf[_rb + l, pl.ds(j * LANES, LANES)]
                            for j in range(DJ))

                    accs = lax.fori_loop(
                        0, L, red,
                        tuple(jnp.zeros((LANES,), jnp.float32)
                              for _ in range(DJ)))
                    out_row = k * CHUNK_ROWS + r
                    for j in range(DJ):
                        pooled[out_row, pl.ds(j * LANES, LANES)] = accs[j] * inv
            return carry

        lax.fori_loop(0, NCHUNK // 2, outer, jnp.int32(0))
        pltpu.sync_copy(pooled, out_hbm.at[pl.ds(base, B_PER_W)])


MLP_BLK = 512


def _mlp_body(x1_ref, x2_ref, w1a_ref, w1b_ref, b1_ref, w2_ref, b2_ref,
              w3_ref, b3_ref, out_ref):
    dn = (((1,), (1,)), ((), ()))
    h = (lax.dot_general(x1_ref[...], w1a_ref[...], dn,
                         preferred_element_type=jnp.float32)
         + lax.dot_general(x2_ref[...], w1b_ref[...], dn,
                           preferred_element_type=jnp.float32)
         + b1_ref[...])
    h = jnp.maximum(h, 0.0)
    h2 = lax.dot_general(h, w2_ref[...], dn,
                         preferred_element_type=jnp.float32) + b2_ref[...]
    h2 = jnp.maximum(h2, 0.0)
    out_ref[...] = lax.dot_general(h2, w3_ref[...], dn,
                                   preferred_element_type=jnp.float32) + b3_ref[...]


def _mlp(x1, x2, W1, b1, W2, b2, W3, b3):
    W1a = W1[:, :D]
    W1b = W1[:, D:]
    return pl.pallas_call(
        _mlp_body,
        grid=(B // MLP_BLK,),
        in_specs=[
            pl.BlockSpec((MLP_BLK, D), lambda i: (i, 0)),
            pl.BlockSpec((MLP_BLK, D), lambda i: (i, 0)),
            pl.BlockSpec((H, D), lambda i: (0, 0)),
            pl.BlockSpec((H, D), lambda i: (0, 0)),
            pl.BlockSpec((1, H), lambda i: (0, 0)),
            pl.BlockSpec((H // 2, H), lambda i: (0, 0)),
            pl.BlockSpec((1, H // 2), lambda i: (0, 0)),
            pl.BlockSpec((A, H // 2), lambda i: (0, 0)),
            pl.BlockSpec((1, A), lambda i: (0, 0)),
        ],
        out_specs=pl.BlockSpec((MLP_BLK, A), lambda i: (i, 0)),
        out_shape=jax.ShapeDtypeStruct((B, A), jnp.float32),
    )(x1, x2, W1a, W1b, b1.reshape(1, H), W2, b2.reshape(1, H // 2),
      W3, b3.reshape(1, A))


def _pad_tokens(tok):
    return jnp.pad(tok.astype(jnp.int32),
                   ((0, 0), (0, L_PAD - L))).reshape(-1)


def kernel(instruction_tokens, scene_tokens, token_emb, scene_emb,
           W1, b1, W2, b2, W3, b3):
    itok = _pad_tokens(instruction_tokens)
    stok = _pad_tokens(scene_tokens)
    i_emb, s_emb = _pool(itok, stok, token_emb, scene_emb)
    return _mlp(i_emb, s_emb, W1, b1, W2, b2, W3, b3)


# speedup vs baseline: 14.2620x; 14.2620x over previous
import functools

import jax
import jax.numpy as jnp
from jax import lax
from jax.experimental import pallas as pl
from jax.experimental.pallas import tpu as pltpu
from jax.experimental.pallas import tpu_sc as plsc

VOCAB = 5000
VOCAB_PAD = 5120
D = 128
H = 256
A = 8
B = 4096
L = 50
L_PAD = 56

_info = plsc.get_sparse_core_info()
NC = _info.num_cores
NS = _info.num_subcores
NW = NC * NS
LANES = _info.num_lanes

B_PER_W = B // NW
CHUNK_ROWS = 2
IDX_PER_CHUNK = CHUNK_ROWS * L_PAD
NCHUNK = B_PER_W // CHUNK_ROWS
DJ = D // LANES
NBUF = 4
STAGE_ROWS = VOCAB_PAD // NS

_sc_mesh = plsc.VectorSubcoreMesh(core_axis_name="c", subcore_axis_name="s")


@functools.partial(
    pl.kernel,
    mesh=_sc_mesh,
    out_type=(
        jax.ShapeDtypeStruct((B, D), jnp.float32),
        jax.ShapeDtypeStruct((B, D), jnp.float32),
    ),
    scratch_types=[
        pltpu.VMEM_SHARED((VOCAB_PAD, D), jnp.float32),
        pltpu.VMEM((B_PER_W * L_PAD,), jnp.int32),
        pltpu.VMEM((NBUF, IDX_PER_CHUNK, D), jnp.float32),
        pltpu.VMEM((B_PER_W, D), jnp.float32),
        pltpu.SemaphoreType.DMA,
        pltpu.SemaphoreType.DMA,
        pltpu.SemaphoreType.DMA,
        pltpu.SemaphoreType.DMA,
    ],
)
def _pool(itok_hbm, stok_hbm, temb_hbm, semb_hbm, iout_hbm, sout_hbm,
          emb_sp, idx_ref, bufs, pooled,
          sem0, sem1, sem2, sem3):
    cid = lax.axis_index("c")
    sid = lax.axis_index("s")
    wid = sid * NC + cid
    base = wid * B_PER_W

    sems = (sem0, sem1, sem2, sem3)
    inv = jnp.float32(1.0 / L)

    for tok_hbm, emb_hbm, out_hbm in ((itok_hbm, temb_hbm, iout_hbm),
                                      (stok_hbm, semb_hbm, sout_hbm)):
        r0 = sid * STAGE_ROWS
        pltpu.sync_copy(emb_hbm.at[pl.ds(r0, STAGE_ROWS)],
                        emb_sp.at[pl.ds(r0, STAGE_ROWS)])
        pltpu.sync_copy(tok_hbm.at[pl.ds(base * L_PAD, B_PER_W * L_PAD)],
                        idx_ref)
        plsc.subcore_barrier()
        def fire(k, bsel):
            off = pl.multiple_of(k * IDX_PER_CHUNK, 8)
            pltpu.async_copy(
                emb_sp.at[idx_ref.at[pl.ds(off, IDX_PER_CHUNK)]],
                bufs.at[bsel], sems[bsel])

        for k0 in range(NBUF):
            fire(k0, k0)

        def outer(kk, carry):
            for bsel in range(NBUF):
                k = kk * NBUF + bsel
                off_k = pl.multiple_of(k * IDX_PER_CHUNK, 8)
                pltpu.make_async_copy(
                    emb_sp.at[idx_ref.at[pl.ds(off_k, IDX_PER_CHUNK)]],
                    bufs.at[bsel], sems[bsel]).wait()

                for r in range(CHUNK_ROWS):
                    rb = r * L_PAD

                    def red(l, accs, _rb=rb, _bsel=bsel):
                        return tuple(
                            accs[j] + bufs[_bsel, _rb + l,
                                           pl.ds(j * LANES, LANES)]
                            for j in range(DJ))

                    accs = lax.fori_loop(
                        0, L, red,
                        tuple(jnp.zeros((LANES,), jnp.float32)
                              for _ in range(DJ)))
                    out_row = k * CHUNK_ROWS + r
                    for j in range(DJ):
                        pooled[out_row, pl.ds(j * LANES, LANES)] = accs[j] * inv

                @pl.when(k + NBUF < NCHUNK)
                def _next(_k=k, _b=bsel):
                    fire(_k + NBUF, _b)
            return carry

        lax.fori_loop(0, NCHUNK // NBUF, outer, jnp.int32(0))
        pltpu.sync_copy(pooled, out_hbm.at[pl.ds(base, B_PER_W)])
        plsc.subcore_barrier()


MLP_BLK = 512


def _mlp_body(x1_ref, x2_ref, w1a_ref, w1b_ref, b1_ref, w2_ref, b2_ref,
              w3_ref, b3_ref, out_ref):
    dn = (((1,), (1,)), ((), ()))
    h = (lax.dot_general(x1_ref[...], w1a_ref[...], dn,
                         preferred_element_type=jnp.float32)
         + lax.dot_general(x2_ref[...], w1b_ref[...], dn,
                           preferred_element_type=jnp.float32)
         + b1_ref[...])
    h = jnp.maximum(h, 0.0)
    h2 = lax.dot_general(h, w2_ref[...], dn,
                         preferred_element_type=jnp.float32) + b2_ref[...]
    h2 = jnp.maximum(h2, 0.0)
    out_ref[...] = lax.dot_general(h2, w3_ref[...], dn,
                                   preferred_element_type=jnp.float32) + b3_ref[...]


def _mlp(x1, x2, W1, b1, W2, b2, W3, b3):
    W1a = W1[:, :D]
    W1b = W1[:, D:]
    return pl.pallas_call(
        _mlp_body,
        grid=(B // MLP_BLK,),
        in_specs=[
            pl.BlockSpec((MLP_BLK, D), lambda i: (i, 0)),
            pl.BlockSpec((MLP_BLK, D), lambda i: (i, 0)),
            pl.BlockSpec((H, D), lambda i: (0, 0)),
            pl.BlockSpec((H, D), lambda i: (0, 0)),
            pl.BlockSpec((1, H), lambda i: (0, 0)),
            pl.BlockSpec((H // 2, H), lambda i: (0, 0)),
            pl.BlockSpec((1, H // 2), lambda i: (0, 0)),
            pl.BlockSpec((A, H // 2), lambda i: (0, 0)),
            pl.BlockSpec((1, A), lambda i: (0, 0)),
        ],
        out_specs=pl.BlockSpec((MLP_BLK, A), lambda i: (i, 0)),
        out_shape=jax.ShapeDtypeStruct((B, A), jnp.float32),
    )(x1, x2, W1a, W1b, b1.reshape(1, H), W2, b2.reshape(1, H // 2),
      W3, b3.reshape(1, A))


def _pad_tokens(tok):
    return jnp.pad(tok.astype(jnp.int32),
                   ((0, 0), (0, L_PAD - L))).reshape(-1)


def kernel(instruction_tokens, scene_tokens, token_emb, scene_emb,
           W1, b1, W2, b2, W3, b3):
    itok = _pad_tokens(instruction_tokens)
    stok = _pad_tokens(scene_tokens)
    temb = jnp.pad(token_emb, ((0, VOCAB_PAD - VOCAB), (0, 0)))
    semb = jnp.pad(scene_emb, ((0, VOCAB_PAD - VOCAB), (0, 0)))
    i_emb, s_emb = _pool(itok, stok, temb, semb)
    return _mlp(i_emb, s_emb, W1, b1, W2, b2, W3, b3)


# speedup vs baseline: 14.7360x; 1.0332x over previous
import functools

import jax
import jax.numpy as jnp
from jax import lax
from jax.experimental import pallas as pl
from jax.experimental.pallas import tpu as pltpu
from jax.experimental.pallas import tpu_sc as plsc

VOCAB = 5000
D = 128
H = 256
A = 8
B = 4096
L = 50
L_PAD = 56

_info = plsc.get_sparse_core_info()
NC = _info.num_cores
NS = _info.num_subcores
NW = NC * NS
LANES = _info.num_lanes

B_PER_W = B // NW
CHUNK_ROWS = 2
IDX_PER_CHUNK = CHUNK_ROWS * L_PAD
NCHUNK = B_PER_W // CHUNK_ROWS
DJ = D // LANES
NBUF = 4
STAGE_ROWS = 312
STAGE_LAST = VOCAB - 15 * STAGE_ROWS

_sc_mesh = plsc.VectorSubcoreMesh(core_axis_name="c", subcore_axis_name="s")


@functools.partial(
    pl.kernel,
    mesh=_sc_mesh,
    out_type=(
        jax.ShapeDtypeStruct((B, D), jnp.float32),
        jax.ShapeDtypeStruct((B, D), jnp.float32),
    ),
    scratch_types=[
        pltpu.VMEM_SHARED((VOCAB, D), jnp.float32),
        pltpu.VMEM((B_PER_W * L_PAD,), jnp.int32),
        pltpu.VMEM((NBUF, IDX_PER_CHUNK, D), jnp.float32),
        pltpu.VMEM((B_PER_W, D), jnp.float32),
        pltpu.SemaphoreType.DMA,
        pltpu.SemaphoreType.DMA,
        pltpu.SemaphoreType.DMA,
        pltpu.SemaphoreType.DMA,
    ],
)
def _pool(itok_hbm, stok_hbm, temb_hbm, semb_hbm, iout_hbm, sout_hbm,
          emb_sp, idx_ref, bufs, pooled, sem0, sem1, sem2, sem3):
    cid = lax.axis_index("c")
    sid = lax.axis_index("s")
    wid = sid * NC + cid
    base = wid * B_PER_W

    sems = (sem0, sem1, sem2, sem3)
    inv = jnp.float32(1.0 / L)

    for tok_hbm, emb_hbm, out_hbm in ((itok_hbm, temb_hbm, iout_hbm),
                                      (stok_hbm, semb_hbm, sout_hbm)):
        @pl.when(sid < NS - 1)
        def _stage_main():
            r0 = sid * STAGE_ROWS
            pltpu.sync_copy(emb_hbm.at[pl.ds(r0, STAGE_ROWS)],
                            emb_sp.at[pl.ds(r0, STAGE_ROWS)])

        @pl.when(sid == NS - 1)
        def _stage_last():
            r0 = (NS - 1) * STAGE_ROWS
            pltpu.sync_copy(emb_hbm.at[pl.ds(r0, STAGE_LAST)],
                            emb_sp.at[pl.ds(r0, STAGE_LAST)])

        pltpu.sync_copy(tok_hbm.at[pl.ds(base * L_PAD, B_PER_W * L_PAD)],
                        idx_ref)
        plsc.subcore_barrier()

        def fire(k, bsel):
            off = pl.multiple_of(k * IDX_PER_CHUNK, 8)
            pltpu.async_copy(
                emb_sp.at[idx_ref.at[pl.ds(off, IDX_PER_CHUNK)]],
                bufs.at[bsel], sems[bsel])

        for k0 in range(NBUF):
            fire(k0, k0)

        def outer(kk, carry):
            for bsel in range(NBUF):
                k = kk * NBUF + bsel
                off_k = pl.multiple_of(k * IDX_PER_CHUNK, 8)
                pltpu.make_async_copy(
                    emb_sp.at[idx_ref.at[pl.ds(off_k, IDX_PER_CHUNK)]],
                    bufs.at[bsel], sems[bsel]).wait()

                for r in range(CHUNK_ROWS):
                    rb = r * L_PAD

                    def red(l, accs, _rb=rb, _bsel=bsel):
                        return tuple(
                            accs[j] + bufs[_bsel, _rb + l,
                                           pl.ds(j * LANES, LANES)]
                            for j in range(DJ))

                    accs = lax.fori_loop(
                        0, L, red,
                        tuple(jnp.zeros((LANES,), jnp.float32)
                              for _ in range(DJ)))
                    out_row = k * CHUNK_ROWS + r
                    for j in range(DJ):
                        pooled[out_row, pl.ds(j * LANES, LANES)] = \
                            accs[j] * inv

                @pl.when(k + NBUF < NCHUNK)
                def _next(_k=k, _b=bsel):
                    fire(_k + NBUF, _b)
            return carry

        lax.fori_loop(0, NCHUNK // NBUF, outer, jnp.int32(0))
        pltpu.sync_copy(pooled, out_hbm.at[pl.ds(base, B_PER_W)])
        plsc.subcore_barrier()


MLP_BLK = 512


def _mlp_body(x1_ref, x2_ref, w1a_ref, w1b_ref, b1_ref, w2_ref, b2_ref,
              w3_ref, b3_ref, out_ref):
    dn = (((1,), (1,)), ((), ()))
    h = (lax.dot_general(x1_ref[...], w1a_ref[...], dn,
                         preferred_element_type=jnp.float32)
         + lax.dot_general(x2_ref[...], w1b_ref[...], dn,
                           preferred_element_type=jnp.float32)
         + b1_ref[...])
    h = jnp.maximum(h, 0.0)
    h2 = lax.dot_general(h, w2_ref[...], dn,
                         preferred_element_type=jnp.float32) + b2_ref[...]
    h2 = jnp.maximum(h2, 0.0)
    out_ref[...] = lax.dot_general(h2, w3_ref[...], dn,
                                   preferred_element_type=jnp.float32) + b3_ref[...]


def _mlp(x1, x2, W1, b1, W2, b2, W3, b3):
    W1a = W1[:, :D]
    W1b = W1[:, D:]
    return pl.pallas_call(
        _mlp_body,
        grid=(B // MLP_BLK,),
        in_specs=[
            pl.BlockSpec((MLP_BLK, D), lambda i: (i, 0)),
            pl.BlockSpec((MLP_BLK, D), lambda i: (i, 0)),
            pl.BlockSpec((H, D), lambda i: (0, 0)),
            pl.BlockSpec((H, D), lambda i: (0, 0)),
            pl.BlockSpec((1, H), lambda i: (0, 0)),
            pl.BlockSpec((H // 2, H), lambda i: (0, 0)),
            pl.BlockSpec((1, H // 2), lambda i: (0, 0)),
            pl.BlockSpec((A, H // 2), lambda i: (0, 0)),
            pl.BlockSpec((1, A), lambda i: (0, 0)),
        ],
        out_specs=pl.BlockSpec((MLP_BLK, A), lambda i: (i, 0)),
        out_shape=jax.ShapeDtypeStruct((B, A), jnp.float32),
    )(x1, x2, W1a, W1b, b1.reshape(1, H), W2, b2.reshape(1, H // 2),
      W3, b3.reshape(1, A))


def _pad_tokens(tok):
    return jnp.pad(tok.astype(jnp.int32),
                   ((0, 0), (0, L_PAD - L))).reshape(-1)


def kernel(instruction_tokens, scene_tokens, token_emb, scene_emb,
           W1, b1, W2, b2, W3, b3):
    itok = _pad_tokens(instruction_tokens)
    stok = _pad_tokens(scene_tokens)
    i_emb, s_emb = _pool(itok, stok, token_emb, scene_emb)
    return _mlp(i_emb, s_emb, W1, b1, W2, b2, W3, b3)


# speedup vs baseline: 16.4062x; 1.1133x over previous
import functools

import jax
import jax.numpy as jnp
from jax import lax
from jax.experimental import pallas as pl
from jax.experimental.pallas import tpu as pltpu
from jax.experimental.pallas import tpu_sc as plsc

VOCAB = 5000
D = 128
H = 256
A = 8
B = 4096
L = 50
GROUP_IDX = 104

_info = plsc.get_sparse_core_info()
NC = _info.num_cores
NS = _info.num_subcores
NW = NC * NS
LANES = _info.num_lanes

B_PER_W = B // NW
CHUNK_ROWS = 2
IDX_PER_CHUNK = GROUP_IDX
NCHUNK = B_PER_W // CHUNK_ROWS
DJ = D // LANES
NBUF = 4
STAGE_ROWS = 312
STAGE_LAST = VOCAB - 15 * STAGE_ROWS

_sc_mesh = plsc.VectorSubcoreMesh(core_axis_name="c", subcore_axis_name="s")


@functools.partial(
    pl.kernel,
    mesh=_sc_mesh,
    out_type=(
        jax.ShapeDtypeStruct((B, D), jnp.float32),
        jax.ShapeDtypeStruct((B, D), jnp.float32),
    ),
    scratch_types=[
        pltpu.VMEM_SHARED((VOCAB, D), jnp.float32),
        pltpu.VMEM((NCHUNK * GROUP_IDX,), jnp.int32),
        pltpu.VMEM((NBUF, IDX_PER_CHUNK, D), jnp.float32),
        pltpu.VMEM((B_PER_W, D), jnp.float32),
        pltpu.SemaphoreType.DMA,
        pltpu.SemaphoreType.DMA,
        pltpu.SemaphoreType.DMA,
        pltpu.SemaphoreType.DMA,
    ],
)
def _pool(itok_hbm, stok_hbm, temb_hbm, semb_hbm, iout_hbm, sout_hbm,
          emb_sp, idx_ref, bufs, pooled, sem0, sem1, sem2, sem3):
    cid = lax.axis_index("c")
    sid = lax.axis_index("s")
    wid = sid * NC + cid
    base = wid * B_PER_W

    sems = (sem0, sem1, sem2, sem3)
    inv = jnp.float32(1.0 / L)

    for tok_hbm, emb_hbm, out_hbm in ((itok_hbm, temb_hbm, iout_hbm),
                                      (stok_hbm, semb_hbm, sout_hbm)):
        @pl.when(sid < NS - 1)
        def _stage_main():
            r0 = sid * STAGE_ROWS
            pltpu.sync_copy(emb_hbm.at[pl.ds(r0, STAGE_ROWS)],
                            emb_sp.at[pl.ds(r0, STAGE_ROWS)])

        @pl.when(sid == NS - 1)
        def _stage_last():
            r0 = (NS - 1) * STAGE_ROWS
            pltpu.sync_copy(emb_hbm.at[pl.ds(r0, STAGE_LAST)],
                            emb_sp.at[pl.ds(r0, STAGE_LAST)])

        pltpu.sync_copy(
            tok_hbm.at[pl.ds(wid * NCHUNK * GROUP_IDX, NCHUNK * GROUP_IDX)],
            idx_ref)
        plsc.subcore_barrier()

        def fire(k, bsel):
            off = pl.multiple_of(k * IDX_PER_CHUNK, 8)
            pltpu.async_copy(
                emb_sp.at[idx_ref.at[pl.ds(off, IDX_PER_CHUNK)]],
                bufs.at[bsel], sems[bsel])

        for k0 in range(NBUF):
            fire(k0, k0)

        def outer(kk, carry):
            for bsel in range(NBUF):
                k = kk * NBUF + bsel
                off_k = pl.multiple_of(k * IDX_PER_CHUNK, 8)
                pltpu.make_async_copy(
                    emb_sp.at[idx_ref.at[pl.ds(off_k, IDX_PER_CHUNK)]],
                    bufs.at[bsel], sems[bsel]).wait()

                for r in range(CHUNK_ROWS):
                    rb = r * L

                    def red(l2, accs, _rb=rb, _bsel=bsel):
                        out = accs
                        for u in range(2):
                            row = _rb + 2 * l2 + u
                            out = tuple(
                                out[j] + bufs[_bsel, row,
                                              pl.ds(j * LANES, LANES)]
                                for j in range(DJ))
                        return out

                    accs = lax.fori_loop(
                        0, L // 2, red,
                        tuple(jnp.zeros((LANES,), jnp.float32)
                              for _ in range(DJ)))
                    out_row = k * CHUNK_ROWS + r
                    for j in range(DJ):
                        pooled[out_row, pl.ds(j * LANES, LANES)] = \
                            accs[j] * inv

                @pl.when(k + NBUF < NCHUNK)
                def _next(_k=k, _b=bsel):
                    fire(_k + NBUF, _b)
            return carry

        lax.fori_loop(0, NCHUNK // NBUF, outer, jnp.int32(0))
        pltpu.sync_copy(pooled, out_hbm.at[pl.ds(base, B_PER_W)])
        plsc.subcore_barrier()


MLP_BLK = 512


def _mlp_body(x1_ref, x2_ref, w1a_ref, w1b_ref, b1_ref, w2_ref, b2_ref,
              w3_ref, b3_ref, out_ref):
    dn = (((1,), (1,)), ((), ()))
    h = (lax.dot_general(x1_ref[...], w1a_ref[...], dn,
                         preferred_element_type=jnp.float32)
         + lax.dot_general(x2_ref[...], w1b_ref[...], dn,
                           preferred_element_type=jnp.float32)
         + b1_ref[...])
    h = jnp.maximum(h, 0.0)
    h2 = lax.dot_general(h, w2_ref[...], dn,
                         preferred_element_type=jnp.float32) + b2_ref[...]
    h2 = jnp.maximum(h2, 0.0)
    out_ref[...] = lax.dot_general(h2, w3_ref[...], dn,
                                   preferred_element_type=jnp.float32) + b3_ref[...]


def _mlp(x1, x2, W1, b1, W2, b2, W3, b3):
    W1a = W1[:, :D]
    W1b = W1[:, D:]
    return pl.pallas_call(
        _mlp_body,
        grid=(B // MLP_BLK,),
        in_specs=[
            pl.BlockSpec((MLP_BLK, D), lambda i: (i, 0)),
            pl.BlockSpec((MLP_BLK, D), lambda i: (i, 0)),
            pl.BlockSpec((H, D), lambda i: (0, 0)),
            pl.BlockSpec((H, D), lambda i: (0, 0)),
            pl.BlockSpec((1, H), lambda i: (0, 0)),
            pl.BlockSpec((H // 2, H), lambda i: (0, 0)),
            pl.BlockSpec((1, H // 2), lambda i: (0, 0)),
            pl.BlockSpec((A, H // 2), lambda i: (0, 0)),
            pl.BlockSpec((1, A), lambda i: (0, 0)),
        ],
        out_specs=pl.BlockSpec((MLP_BLK, A), lambda i: (i, 0)),
        out_shape=jax.ShapeDtypeStruct((B, A), jnp.float32),
    )(x1, x2, W1a, W1b, b1.reshape(1, H), W2, b2.reshape(1, H // 2),
      W3, b3.reshape(1, A))


def _pad_tokens(tok):
    pairs = tok.astype(jnp.int32).reshape(B // 2, 2 * L)
    return jnp.pad(pairs, ((0, 0), (0, GROUP_IDX - 2 * L))).reshape(-1)


def kernel(instruction_tokens, scene_tokens, token_emb, scene_emb,
           W1, b1, W2, b2, W3, b3):
    itok = _pad_tokens(instruction_tokens)
    stok = _pad_tokens(scene_tokens)
    i_emb, s_emb = _pool(itok, stok, token_emb, scene_emb)
    return _mlp(i_emb, s_emb, W1, b1, W2, b2, W3, b3)


# speedup vs baseline: 16.8291x; 1.0258x over previous
import functools

import jax
import jax.numpy as jnp
from jax import lax
from jax.experimental import pallas as pl
from jax.experimental.pallas import tpu as pltpu
from jax.experimental.pallas import tpu_sc as plsc

VOCAB = 5000
D = 128
H = 256
A = 8
B = 4096
L = 50
GROUP_IDX = 104

_info = plsc.get_sparse_core_info()
NC = _info.num_cores
NS = _info.num_subcores
NW = NC * NS
LANES = _info.num_lanes

B_PER_W = B // NW
CHUNK_ROWS = 2
IDX_PER_CHUNK = GROUP_IDX
NCHUNK = B_PER_W // CHUNK_ROWS
DJ = D // LANES
NBUF = 4
STAGE_ROWS = 312
STAGE_LAST = VOCAB - 15 * STAGE_ROWS

_sc_mesh = plsc.VectorSubcoreMesh(core_axis_name="c", subcore_axis_name="s")


@functools.partial(
    pl.kernel,
    mesh=_sc_mesh,
    out_type=(
        jax.ShapeDtypeStruct((B, D), jnp.float32),
        jax.ShapeDtypeStruct((B, D), jnp.float32),
    ),
    scratch_types=[
        pltpu.VMEM_SHARED((VOCAB, D), jnp.float32),
        pltpu.VMEM((NCHUNK * GROUP_IDX,), jnp.int32),
        pltpu.VMEM((NBUF, IDX_PER_CHUNK, D), jnp.float32),
        pltpu.VMEM((B_PER_W, D), jnp.float32),
        pltpu.SemaphoreType.DMA,
        pltpu.SemaphoreType.DMA,
        pltpu.SemaphoreType.DMA,
        pltpu.SemaphoreType.DMA,
    ],
)
def _pool(itok_hbm, stok_hbm, temb_hbm, semb_hbm, iout_hbm, sout_hbm,
          emb_sp, idx_ref, bufs, pooled, sem0, sem1, sem2, sem3):
    cid = lax.axis_index("c")
    sid = lax.axis_index("s")
    wid = sid * NC + cid
    base = wid * B_PER_W

    sems = (sem0, sem1, sem2, sem3)
    inv = jnp.float32(1.0 / L)

    for tok_hbm, emb_hbm, out_hbm in ((itok_hbm, temb_hbm, iout_hbm),
                                      (stok_hbm, semb_hbm, sout_hbm)):
        @pl.when(sid < NS - 1)
        def _stage_main():
            r0 = sid * STAGE_ROWS
            pltpu.sync_copy(emb_hbm.at[pl.ds(r0, STAGE_ROWS)],
                            emb_sp.at[pl.ds(r0, STAGE_ROWS)])

        @pl.when(sid == NS - 1)
        def _stage_last():
            r0 = (NS - 1) * STAGE_ROWS
            pltpu.sync_copy(emb_hbm.at[pl.ds(r0, STAGE_LAST)],
                            emb_sp.at[pl.ds(r0, STAGE_LAST)])

        pltpu.sync_copy(
            tok_hbm.at[pl.ds(wid * NCHUNK * GROUP_IDX, NCHUNK * GROUP_IDX)],
            idx_ref)
        plsc.subcore_barrier()

        def fire(k, bsel):
            off = pl.multiple_of(k * IDX_PER_CHUNK, 8)
            pltpu.async_copy(
                emb_sp.at[idx_ref.at[pl.ds(off, IDX_PER_CHUNK)]],
                bufs.at[bsel], sems[bsel])

        for k0 in range(NBUF):
            fire(k0, k0)

        def outer(kk, carry):
            for bsel in range(NBUF):
                k = kk * NBUF + bsel
                off_k = pl.multiple_of(k * IDX_PER_CHUNK, 8)
                pltpu.make_async_copy(
                    emb_sp.at[idx_ref.at[pl.ds(off_k, IDX_PER_CHUNK)]],
                    bufs.at[bsel], sems[bsel]).wait()

                for r in range(CHUNK_ROWS):
                    rb = r * L

                    def red(l2, accs, _rb=rb, _bsel=bsel):
                        out = accs
                        for u in range(2):
                            row = _rb + 2 * l2 + u
                            out = tuple(
                                out[j] + bufs[_bsel, row,
                                              pl.ds(j * LANES, LANES)]
                                for j in range(DJ))
                        return out

                    accs = lax.fori_loop(
                        0, L // 2, red,
                        tuple(jnp.zeros((LANES,), jnp.float32)
                              for _ in range(DJ)))
                    out_row = k * CHUNK_ROWS + r
                    for j in range(DJ):
                        pooled[out_row, pl.ds(j * LANES, LANES)] = \
                            accs[j] * inv

                @pl.when(k + NBUF < NCHUNK)
                def _next(_k=k, _b=bsel):
                    fire(_k + NBUF, _b)
            return carry

        lax.fori_loop(0, NCHUNK // NBUF, outer, jnp.int32(0))
        pltpu.sync_copy(pooled, out_hbm.at[pl.ds(base, B_PER_W)])
        plsc.subcore_barrier()


MLP_BLK = 2048


def _mlp_body(x1_ref, x2_ref, w1a_ref, w1b_ref, b1_ref, w2_ref, b2_ref,
              w3_ref, b3_ref, out_ref):
    dn = (((1,), (1,)), ((), ()))
    h = (lax.dot_general(x1_ref[...], w1a_ref[...], dn,
                         preferred_element_type=jnp.float32)
         + lax.dot_general(x2_ref[...], w1b_ref[...], dn,
                           preferred_element_type=jnp.float32)
         + b1_ref[...])
    h = jnp.maximum(h, 0.0)
    h2 = lax.dot_general(h, w2_ref[...], dn,
                         preferred_element_type=jnp.float32) + b2_ref[...]
    h2 = jnp.maximum(h2, 0.0)
    out_ref[...] = lax.dot_general(h2, w3_ref[...], dn,
                                   preferred_element_type=jnp.float32) + b3_ref[...]


def _mlp(x1, x2, W1, b1, W2, b2, W3, b3):
    W1a = W1[:, :D]
    W1b = W1[:, D:]
    return pl.pallas_call(
        _mlp_body,
        grid=(B // MLP_BLK,),
        in_specs=[
            pl.BlockSpec((MLP_BLK, D), lambda i: (i, 0)),
            pl.BlockSpec((MLP_BLK, D), lambda i: (i, 0)),
            pl.BlockSpec((H, D), lambda i: (0, 0)),
            pl.BlockSpec((H, D), lambda i: (0, 0)),
            pl.BlockSpec((1, H), lambda i: (0, 0)),
            pl.BlockSpec((H // 2, H), lambda i: (0, 0)),
            pl.BlockSpec((1, H // 2), lambda i: (0, 0)),
            pl.BlockSpec((A, H // 2), lambda i: (0, 0)),
            pl.BlockSpec((1, A), lambda i: (0, 0)),
        ],
        out_specs=pl.BlockSpec((MLP_BLK, A), lambda i: (i, 0)),
        out_shape=jax.ShapeDtypeStruct((B, A), jnp.float32),
    )(x1, x2, W1a, W1b, b1.reshape(1, H), W2, b2.reshape(1, H // 2),
      W3, b3.reshape(1, A))


def _pad_tokens(tok):
    pairs = tok.astype(jnp.int32).reshape(B // 2, 2 * L)
    return jnp.pad(pairs, ((0, 0), (0, GROUP_IDX - 2 * L))).reshape(-1)


def kernel(instruction_tokens, scene_tokens, token_emb, scene_emb,
           W1, b1, W2, b2, W3, b3):
    itok = _pad_tokens(instruction_tokens)
    stok = _pad_tokens(scene_tokens)
    i_emb, s_emb = _pool(itok, stok, token_emb, scene_emb)
    return _mlp(i_emb, s_emb, W1, b1, W2, b2, W3, b3)


# speedup vs baseline: 17.0127x; 1.0109x over previous
import functools

import jax
import jax.numpy as jnp
from jax import lax
from jax.experimental import pallas as pl
from jax.experimental.pallas import tpu as pltpu
from jax.experimental.pallas import tpu_sc as plsc

VOCAB = 5000
D = 128
H = 256
A = 8
B = 4096
L = 50

_info = plsc.get_sparse_core_info()
NC = _info.num_cores
NS = _info.num_subcores
NW = NC * NS
LANES = _info.num_lanes

B_PER_W = B // NW
IDX_PER_W = B_PER_W * L
NPAIR = B_PER_W // 4
CH_A = 104
CH_B = 96
DJ = D // LANES
NBUF = 4
STAGE_ROWS = 312
STAGE_LAST = VOCAB - 15 * STAGE_ROWS

_sc_mesh = plsc.VectorSubcoreMesh(core_axis_name="c", subcore_axis_name="s")


@functools.partial(
    pl.kernel,
    mesh=_sc_mesh,
    out_type=(
        jax.ShapeDtypeStruct((B, D), jnp.float32),
        jax.ShapeDtypeStruct((B, D), jnp.float32),
    ),
    scratch_types=[
        pltpu.VMEM_SHARED((VOCAB, D), jnp.float32),
        pltpu.VMEM((IDX_PER_W,), jnp.int32),
        pltpu.VMEM((NBUF, CH_A, D), jnp.float32),
        pltpu.VMEM((B_PER_W, D), jnp.float32),
        pltpu.SemaphoreType.DMA,
        pltpu.SemaphoreType.DMA,
        pltpu.SemaphoreType.DMA,
        pltpu.SemaphoreType.DMA,
    ],
)
def _pool(itok_hbm, stok_hbm, temb_hbm, semb_hbm, iout_hbm, sout_hbm,
          emb_sp, idx_ref, bufs, pooled, sem0, sem1, sem2, sem3):
    cid = lax.axis_index("c")
    sid = lax.axis_index("s")
    wid = sid * NC + cid
    base = wid * B_PER_W

    sems = (sem0, sem1, sem2, sem3)
    inv = jnp.float32(1.0 / L)

    for tok_hbm, emb_hbm, out_hbm in ((itok_hbm, temb_hbm, iout_hbm),
                                      (stok_hbm, semb_hbm, sout_hbm)):
        @pl.when(sid < NS - 1)
        def _stage_main():
            r0 = sid * STAGE_ROWS
            pltpu.sync_copy(emb_hbm.at[pl.ds(r0, STAGE_ROWS)],
                            emb_sp.at[pl.ds(r0, STAGE_ROWS)])

        @pl.when(sid == NS - 1)
        def _stage_last():
            r0 = (NS - 1) * STAGE_ROWS
            pltpu.sync_copy(emb_hbm.at[pl.ds(r0, STAGE_LAST)],
                            emb_sp.at[pl.ds(r0, STAGE_LAST)])

        pltpu.sync_copy(tok_hbm.at[pl.ds(wid * IDX_PER_W, IDX_PER_W)],
                        idx_ref)
        plsc.subcore_barrier()

        def fire_pair(m, sA, sB):
            offA = pl.multiple_of(m * 200, 8)
            offB = pl.multiple_of(m * 200 + CH_A, 8)
            pltpu.async_copy(
                emb_sp.at[idx_ref.at[pl.ds(offA, CH_A)]],
                bufs.at[sA], sems[sA])
            pltpu.async_copy(
                emb_sp.at[idx_ref.at[pl.ds(offB, CH_B)]],
                bufs.at[sB, pl.ds(0, CH_B)], sems[sB])

        def wait_pair(m, sA, sB):
            offA = pl.multiple_of(m * 200, 8)
            offB = pl.multiple_of(m * 200 + CH_A, 8)
            pltpu.make_async_copy(
                emb_sp.at[idx_ref.at[pl.ds(offA, CH_A)]],
                bufs.at[sA], sems[sA]).wait()
            pltpu.make_async_copy(
                emb_sp.at[idx_ref.at[pl.ds(offB, CH_B)]],
                bufs.at[sB, pl.ds(0, CH_B)], sems[sB]).wait()

        fire_pair(0, 0, 1)
        fire_pair(1, 2, 3)

        def sum_rows(segs):
            accs = tuple(jnp.zeros((LANES,), jnp.float32)
                         for _ in range(DJ))
            for bsel, start, count in segs:
                def red(l2, a, _s=start, _b=bsel):
                    out = a
                    for u in range(2):
                        row = _s + 2 * l2 + u
                        out = tuple(
                            out[j] + bufs[_b, row, pl.ds(j * LANES, LANES)]
                            for j in range(DJ))
                    return out
                accs = lax.fori_loop(0, count // 2, red, accs)
            return accs

        def outer(mm, carry):
            for mpar in range(2):
                m = mm * 2 + mpar
                sA, sB = 2 * mpar, 2 * mpar + 1
                wait_pair(m, sA, sB)

                row_segs = (
                    ((sA, 0, 50),),
                    ((sA, 50, 50),),
                    ((sA, 100, 4), (sB, 0, 46)),
                    ((sB, 46, 50),),
                )
                for r in range(4):
                    accs = sum_rows(row_segs[r])
                    out_row = m * 4 + r
                    for j in range(DJ):
                        pooled[out_row, pl.ds(j * LANES, LANES)] = \
                            accs[j] * inv

                @pl.when(m + 2 < NPAIR)
                def _next(_m=m, _sA=sA, _sB=sB):
                    fire_pair(_m + 2, _sA, _sB)
            return carry

        lax.fori_loop(0, NPAIR // 2, outer, jnp.int32(0))
        pltpu.sync_copy(pooled, out_hbm.at[pl.ds(base, B_PER_W)])
        plsc.subcore_barrier()


MLP_BLK = 2048


def _mlp_body(x1_ref, x2_ref, w1a_ref, w1b_ref, b1_ref, w2_ref, b2_ref,
              w3_ref, b3_ref, out_ref):
    dn = (((1,), (1,)), ((), ()))
    h = (lax.dot_general(x1_ref[...], w1a_ref[...], dn,
                         preferred_element_type=jnp.float32)
         + lax.dot_general(x2_ref[...], w1b_ref[...], dn,
                           preferred_element_type=jnp.float32)
         + b1_ref[...])
    h = jnp.maximum(h, 0.0)
    h2 = lax.dot_general(h, w2_ref[...], dn,
                         preferred_element_type=jnp.float32) + b2_ref[...]
    h2 = jnp.maximum(h2, 0.0)
    out_ref[...] = lax.dot_general(h2, w3_ref[...], dn,
                                   preferred_element_type=jnp.float32) + b3_ref[...]


def _mlp(x1, x2, W1, b1, W2, b2, W3, b3):
    W1a = W1[:, :D]
    W1b = W1[:, D:]
    return pl.pallas_call(
        _mlp_body,
        grid=(B // MLP_BLK,),
        in_specs=[
            pl.BlockSpec((MLP_BLK, D), lambda i: (i, 0)),
            pl.BlockSpec((MLP_BLK, D), lambda i: (i, 0)),
            pl.BlockSpec((H, D), lambda i: (0, 0)),
            pl.BlockSpec((H, D), lambda i: (0, 0)),
            pl.BlockSpec((1, H), lambda i: (0, 0)),
            pl.BlockSpec((H // 2, H), lambda i: (0, 0)),
            pl.BlockSpec((1, H // 2), lambda i: (0, 0)),
            pl.BlockSpec((A, H // 2), lambda i: (0, 0)),
            pl.BlockSpec((1, A), lambda i: (0, 0)),
        ],
        out_specs=pl.BlockSpec((MLP_BLK, A), lambda i: (i, 0)),
        out_shape=jax.ShapeDtypeStruct((B, A), jnp.float32),
    )(x1, x2, W1a, W1b, b1.reshape(1, H), W2, b2.reshape(1, H // 2),
      W3, b3.reshape(1, A))


def _pad_tokens(tok):
    return tok.astype(jnp.int32).reshape(-1)


def kernel(instruction_tokens, scene_tokens, token_emb, scene_emb,
           W1, b1, W2, b2, W3, b3):
    itok = _pad_tokens(instruction_tokens)
    stok = _pad_tokens(scene_tokens)
    i_emb, s_emb = _pool(itok, stok, token_emb, scene_emb)
    return _mlp(i_emb, s_emb, W1, b1, W2, b2, W3, b3)
